# baseline (device time: 17163 ns/iter reference)
import jax
import jax.numpy as jnp
from jax import lax
from jax.experimental import pallas as pl
from jax.experimental.pallas import tpu as pltpu

QROWS = 128
CROWS = 64
QC = QROWS // CROWS


def kernel(x, pi):
    def body(pi_ref, x_ref, out_ref,
             xs, xr, ys1, yr1, zs1, zr1, ys2, yr2, zs2, zr2):
        my_x = lax.axis_index("x")
        my_y = lax.axis_index("y")
        my_z = lax.axis_index("z")
        dst_x = pi_ref[my_x]
        other_y = 1 - my_y
        p = lax.rem(my_z, 2)
        z_nb = my_z + 1 - 2 * p

        q_me = 2 * my_y + p
        q_y = 2 * other_y + p
        q_z = 2 * my_y + (1 - p)
        q_d = 2 * other_y + (1 - p)

        def rows(q, k):
            return pl.ds(q * QROWS + k * CROWS, CROWS)

        def copy(src_ref, dst_ref, send_sem, recv_sem, device_id):
            return pltpu.make_async_remote_copy(
                src_ref=src_ref, dst_ref=dst_ref,
                send_sem=send_sem, recv_sem=recv_sem,
                device_id=device_id, device_id_type=pl.DeviceIdType.MESH,
            )

        x_dev = (dst_x, my_y, my_z)
        y_dev = (my_x, other_y, my_z)
        z_dev = (my_x, my_y, z_nb)

        barrier_sem = pltpu.get_barrier_semaphore()
        for dev in (x_dev, y_dev, z_dev):
            pl.semaphore_signal(
                barrier_sem, inc=1,
                device_id=dev, device_id_type=pl.DeviceIdType.MESH,
            )
        pl.semaphore_wait(barrier_sem, 3)

        x_rdmas = []
        for k in range(QC):
            r = copy(x_ref.at[0, rows(q_me, k), :],
                     out_ref.at[0, rows(q_me, k), :],
                     xs.at[k], xr.at[k], x_dev)
            r.start()
            x_rdmas.append(r)

        y1_rdmas, z1_rdmas = [], []
        for k in range(QC):
            x_rdmas[k].wait_recv()
            fy = copy(out_ref.at[0, rows(q_me, k), :],
                      out_ref.at[0, rows(q_me, k), :],
                      ys1.at[k], yr1.at[k], y_dev)
            fz = copy(out_ref.at[0, rows(q_me, k), :],
                      out_ref.at[0, rows(q_me, k), :],
                      zs1.at[k], zr1.at[k], z_dev)
            fy.start()
            fz.start()
            y1_rdmas.append(fy)
            z1_rdmas.append(fz)

        yr1_w0 = copy(out_ref.at[0, rows(q_y, 0), :],
                      out_ref.at[0, rows(q_y, 0), :],
                      ys1.at[0], yr1.at[0], y_dev)
        yr1_w0.wait_recv()
        s2z = copy(out_ref.at[0, rows(q_y, 0), :],
                   out_ref.at[0, rows(q_y, 0), :],
                   zs2.at[0], zr2.at[0], z_dev)
        s2z.start()

        zr1_w1 = copy(out_ref.at[0, rows(q_z, 1), :],
                      out_ref.at[0, rows(q_z, 1), :],
                      zs1.at[1], zr1.at[1], z_dev)
        zr1_w1.wait_recv()
        s2y = copy(out_ref.at[0, rows(q_z, 1), :],
                   out_ref.at[0, rows(q_z, 1), :],
                   ys2.at[0], yr2.at[0], y_dev)
        s2y.start()

        copy(out_ref.at[0, rows(q_y, 1), :],
             out_ref.at[0, rows(q_y, 1), :],
             ys1.at[1], yr1.at[1], y_dev).wait_recv()
        copy(out_ref.at[0, rows(q_z, 0), :],
             out_ref.at[0, rows(q_z, 0), :],
             zs1.at[0], zr1.at[0], z_dev).wait_recv()

        copy(out_ref.at[0, rows(q_d, 0), :],
             out_ref.at[0, rows(q_d, 0), :],
             zs2.at[0], zr2.at[0], z_dev).wait_recv()
        copy(out_ref.at[0, rows(q_d, 1), :],
             out_ref.at[0, rows(q_d, 1), :],
             ys2.at[0], yr2.at[0], y_dev).wait_recv()

        for k in range(QC):
            x_rdmas[k].wait_send()
            y1_rdmas[k].wait_send()
            z1_rdmas[k].wait_send()
        s2z.wait_send()
        s2y.wait_send()

    return pl.pallas_call(
        body,
        out_shape=jax.ShapeDtypeStruct(x.shape, x.dtype),
        in_specs=[
            pl.BlockSpec(memory_space=pltpu.SMEM),
            pl.BlockSpec(memory_space=pl.ANY),
        ],
        out_specs=pl.BlockSpec(memory_space=pl.ANY),
        scratch_shapes=[
            pltpu.SemaphoreType.DMA((QC,)),
            pltpu.SemaphoreType.DMA((QC,)),
            pltpu.SemaphoreType.DMA((QC,)),
            pltpu.SemaphoreType.DMA((QC,)),
            pltpu.SemaphoreType.DMA((QC,)),
            pltpu.SemaphoreType.DMA((QC,)),
            pltpu.SemaphoreType.DMA((1,)),
            pltpu.SemaphoreType.DMA((1,)),
            pltpu.SemaphoreType.DMA((1,)),
            pltpu.SemaphoreType.DMA((1,)),
        ],
        compiler_params=pltpu.CompilerParams(collective_id=0),
    )(pi, x)


# device time: 15229 ns/iter; 1.1270x vs baseline; 1.1270x over previous
import jax
import jax.numpy as jnp
from jax import lax
from jax.experimental import pallas as pl
from jax.experimental.pallas import tpu as pltpu

ROWS = 512
CROWS = 64
NC = 5
NF = 3


def kernel(x, pi):
    def body(pi_ref, x_ref, out_ref, xs, xr, ys, yr):
        my_x = lax.axis_index("x")
        my_y = lax.axis_index("y")
        my_z = lax.axis_index("z")
        dst_x = pi_ref[my_x]
        other_y = 1 - my_y

        top = ROWS - CROWS
        base = my_y * top
        sign = 1 - 2 * my_y
        nb_base = other_y * top
        nb_sign = 1 - 2 * other_y

        def copy(src_ref, dst_ref, send_sem, recv_sem, device_id):
            return pltpu.make_async_remote_copy(
                src_ref=src_ref, dst_ref=dst_ref,
                send_sem=send_sem, recv_sem=recv_sem,
                device_id=device_id, device_id_type=pl.DeviceIdType.MESH,
            )

        x_dev = (dst_x, my_y, my_z)
        y_dev = (my_x, other_y, my_z)

        barrier_sem = pltpu.get_barrier_semaphore()
        for dev in (x_dev, y_dev):
            pl.semaphore_signal(
                barrier_sem, inc=1,
                device_id=dev, device_id_type=pl.DeviceIdType.MESH,
            )
        pl.semaphore_wait(barrier_sem, 2)

        x_rdmas = []
        for k in range(NC):
            rows = pl.ds(base + sign * (k * CROWS), CROWS)
            r = copy(x_ref.at[0, rows, :], out_ref.at[0, rows, :],
                     xs.at[k], xr.at[k], x_dev)
            r.start()
            x_rdmas.append(r)

        y_rdmas = []
        for k in range(NF):
            x_rdmas[k].wait_recv()
            rows = pl.ds(base + sign * (k * CROWS), CROWS)
            f = copy(out_ref.at[0, rows, :], out_ref.at[0, rows, :],
                     ys.at[k], yr.at[k], y_dev)
            f.start()
            y_rdmas.append(f)

        for k in range(NF, NC):
            x_rdmas[k].wait_recv()

        for k in range(NF):
            rows = pl.ds(nb_base + nb_sign * (k * CROWS), CROWS)
            copy(out_ref.at[0, rows, :], out_ref.at[0, rows, :],
                 ys.at[k], yr.at[k], y_dev).wait_recv()

        for k in range(NC):
            x_rdmas[k].wait_send()
        for k in range(NF):
            y_rdmas[k].wait_send()

    return pl.pallas_call(
        body,
        out_shape=jax.ShapeDtypeStruct(x.shape, x.dtype),
        in_specs=[
            pl.BlockSpec(memory_space=pltpu.SMEM),
            pl.BlockSpec(memory_space=pl.ANY),
        ],
        out_specs=pl.BlockSpec(memory_space=pl.ANY),
        scratch_shapes=[
            pltpu.SemaphoreType.DMA((NC,)),
            pltpu.SemaphoreType.DMA((NC,)),
            pltpu.SemaphoreType.DMA((NF,)),
            pltpu.SemaphoreType.DMA((NF,)),
        ],
        compiler_params=pltpu.CompilerParams(collective_id=0),
    )(pi, x)
